# baseline (speedup 1.0000x reference)
import functools

import jax
import jax.numpy as jnp
from jax import lax
from jax.experimental import pallas as pl
from jax.experimental.pallas import tpu as pltpu
from jax.experimental.pallas import tpu_sc as plsc

N_DOMAIN = 26
VOCAB = 100000
DIM = 64
BATCH = 16384
NW = 32
N_ROWS = N_DOMAIN * DIM
R_PER_W = N_ROWS // NW
OCHUNK = 4096
NOB = BATCH // OCHUNK
LANES = 16

_mesh = plsc.VectorSubcoreMesh(core_axis_name="c", subcore_axis_name="s")


@functools.partial(
    pl.kernel,
    mesh=_mesh,
    compiler_params=pltpu.CompilerParams(needs_layout_passes=False),
    out_type=jax.ShapeDtypeStruct((N_DOMAIN, DIM, BATCH), jnp.float32),
    scratch_types=[
        pltpu.VMEM((VOCAB,), jnp.float32),
        pltpu.VMEM((BATCH,), jnp.int32),
        pltpu.VMEM((2, OCHUNK), jnp.float32),
        pltpu.VMEM((64 * 32,), jnp.float32),
        pltpu.SemaphoreType.DMA((2,)),
        pltpu.SemaphoreType.DMA((2,)),
    ],
)
def _gather_kernel(cond_hbm, tables_hbm, tail_hbm, out_hbm, row_v, idx_v,
                   obuf, tail_v, lsem, osems):
    wid = lax.axis_index("s") * 2 + lax.axis_index("c")
    r0 = wid * R_PER_W
    i0 = r0 // DIM

    HALF_A = 50048
    HALF_B = 49920
    TAIL = 32
    TAIL_START = VOCAB - TAIL

    def load_idx(i):
        pltpu.async_copy(cond_hbm.at[i], idx_v, lsem.at[0]).wait()
        pltpu.async_copy(tail_hbm.at[i], tail_v, lsem.at[0]).wait()

    def do_row(r, carry):
        i = r // DIM
        d = r % DIM
        ca = pltpu.async_copy(tables_hbm.at[i, d].at[pl.ds(0, HALF_A)],
                              row_v.at[pl.ds(0, HALF_A)], lsem.at[0])
        cb = pltpu.async_copy(tables_hbm.at[i, d].at[pl.ds(HALF_A, HALF_B)],
                              row_v.at[pl.ds(HALF_A, HALF_B)], lsem.at[1])
        for t in range(TAIL // LANES):
            row_v[pl.ds(TAIL_START + t * LANES, LANES)] = (
                tail_v[pl.ds(d * TAIL + t * LANES, LANES)])
        ca.wait()
        cb.wait()
        for c in range(NOB):
            slot = c % 2
            @pl.when(jnp.logical_or(r > r0, c >= 2))
            def _():
                pltpu.make_async_copy(
                    obuf.at[slot],
                    out_hbm.at[i, d, pl.ds(0, OCHUNK)],
                    osems.at[slot],
                ).wait()

            def gather16(k, _):
                idxv = idx_v[pl.ds(c * OCHUNK + k * LANES, LANES)]
                vals = plsc.load_gather(row_v, [idxv])
                obuf[slot, pl.ds(k * LANES, LANES)] = vals
                return _

            lax.fori_loop(0, OCHUNK // LANES, gather16, 0, unroll=16)
            pltpu.async_copy(
                obuf.at[slot],
                out_hbm.at[i, d, pl.ds(c * OCHUNK, OCHUNK)],
                osems.at[slot],
            )
        return carry

    seg_end = jnp.minimum((i0 + 1) * DIM, r0 + R_PER_W)
    load_idx(i0)
    lax.fori_loop(r0, seg_end, do_row, 0)

    @pl.when(seg_end < r0 + R_PER_W)
    def _():
        load_idx(i0 + 1)
        lax.fori_loop(seg_end, r0 + R_PER_W, do_row, 0)

    for slot in range(2):
        pltpu.make_async_copy(
            obuf.at[slot],
            out_hbm.at[0, 0, pl.ds(0, OCHUNK)],
            osems.at[slot],
        ).wait()


def kernel(conditions, tables):
    cond_t = conditions.astype(jnp.int32).T
    tables_t = jnp.transpose(tables, (0, 2, 1))
    tails = jnp.transpose(tables[:, VOCAB - 32:, :], (0, 2, 1))
    tails = tails.reshape(N_DOMAIN, 64 * 32)
    out = _gather_kernel(cond_t, tables_t, tails)
    out = out.reshape(N_DOMAIN, 8, 8, BATCH)
    return jnp.transpose(out, (3, 0, 1, 2))


# speedup vs baseline: 1.0085x; 1.0085x over previous
import functools

import jax
import jax.numpy as jnp
from jax import lax
from jax.experimental import pallas as pl
from jax.experimental.pallas import tpu as pltpu
from jax.experimental.pallas import tpu_sc as plsc

N_DOMAIN = 26
VOCAB = 100000
DIM = 64
BATCH = 16384
NW = 32
N_ROWS = N_DOMAIN * DIM
R_PER_W = N_ROWS // NW
OCHUNK = 4096
NOB = BATCH // OCHUNK
LANES = 16

_mesh = plsc.VectorSubcoreMesh(core_axis_name="c", subcore_axis_name="s")


@functools.partial(
    pl.kernel,
    mesh=_mesh,
    compiler_params=pltpu.CompilerParams(needs_layout_passes=False),
    out_type=jax.ShapeDtypeStruct((N_DOMAIN, DIM, BATCH), jnp.float32),
    scratch_types=[
        pltpu.VMEM((VOCAB,), jnp.float32),
        pltpu.VMEM((BATCH,), jnp.int32),
        pltpu.VMEM((2, OCHUNK), jnp.float32),
        pltpu.SemaphoreType.DMA,
        pltpu.SemaphoreType.DMA((2,)),
    ],
)
def _gather_kernel(cond_hbm, tables_hbm, out_hbm, row_v, idx_v, obuf, lsem,
                   osems):
    wid = lax.axis_index("s") * 2 + lax.axis_index("c")
    grp = wid // 8
    lane = wid % 8
    t0 = grp * R_PER_W

    def load_idx(i):
        pltpu.async_copy(cond_hbm.at[i], idx_v, lsem).wait()

    def do_row(t, carry):
        band = t0 + t
        i = band // 8
        d = (band % 8) * 8 + lane
        r = t0 + t
        r0 = t0

        @pl.when(jnp.logical_and(t > 0, band % 8 == 0))
        def _():
            load_idx(i)

        pltpu.async_copy(tables_hbm.at[i, d], row_v, lsem).wait()
        for c in range(NOB):
            slot = c % 2
            @pl.when(jnp.logical_or(r > r0, c >= 2))
            def _():
                pltpu.make_async_copy(
                    obuf.at[slot],
                    out_hbm.at[i, d, pl.ds(0, OCHUNK)],
                    osems.at[slot],
                ).wait()

            def gather16(k, _):
                idxv = idx_v[pl.ds(c * OCHUNK + k * LANES, LANES)]
                vals = plsc.load_gather(row_v, [idxv])
                obuf[slot, pl.ds(k * LANES, LANES)] = vals
                return _

            lax.fori_loop(0, OCHUNK // LANES, gather16, 0, unroll=16)
            pltpu.async_copy(
                obuf.at[slot],
                out_hbm.at[i, d, pl.ds(c * OCHUNK, OCHUNK)],
                osems.at[slot],
            )
        return carry

    load_idx(t0 // 8)
    lax.fori_loop(0, R_PER_W, do_row, 0)

    for slot in range(2):
        pltpu.make_async_copy(
            obuf.at[slot],
            out_hbm.at[0, 0, pl.ds(0, OCHUNK)],
            osems.at[slot],
        ).wait()


def kernel(conditions, tables):
    cond_t = conditions.astype(jnp.int32).T
    tables_t = jnp.transpose(tables, (0, 2, 1))
    out = _gather_kernel(cond_t, tables_t)
    out = out.reshape(N_DOMAIN, 8, 8, BATCH)
    return jnp.transpose(out, (3, 0, 1, 2))
